# Initial kernel scaffold; baseline (speedup 1.0000x reference)
import jax
import jax.numpy as jnp
from jax.experimental import pallas as pl


def kernel(in_feat, edge_index, W0, b0, W1, b1):
    raise NotImplementedError("write your pallas kernel here")


# speedup vs baseline: 49.8277x; 49.8277x over previous
import functools

import jax
import jax.numpy as jnp
from jax import lax
from jax.experimental import pallas as pl
from jax.experimental.pallas import tpu as pltpu
from jax.experimental.pallas import tpu_sc as plsc

_NC = 2
_NS = 16
_NW = _NC * _NS
_ROW = 128
_K = 8
_TARGET = 1


def _mesh():
    return plsc.VectorSubcoreMesh(
        core_axis_name="c", subcore_axis_name="s",
        num_cores=_NC, num_subcores=_NS)


def _edge_stats_body(n_acc, stripe, chunks, rows_per_worker,
                     src_hbm, dst_hbm, out_do, out_di, out_k,
                     src2d, dst2d, ones_v, mrow, zobuf,
                     acc_do, acc_di, acc_k):
    cid = lax.axis_index("c")
    sid = lax.axis_index("s")
    wid = sid * _NC + cid

    for t in range(_ROW // 16):
        ones_v[pl.ds(t * 16, 16)] = jnp.ones((16,), jnp.float32)

    def _zero(i, carry):
        zobuf[pl.ds(i * 16, 16)] = jnp.zeros((16,), jnp.float32)
        return carry
    lax.fori_loop(0, stripe // 16, _zero, 0)

    off = pl.multiple_of(sid * stripe, 128)
    pltpu.sync_copy(zobuf, acc_do.at[pl.ds(off, stripe)])
    pltpu.sync_copy(zobuf, acc_di.at[pl.ds(off, stripe)])
    pltpu.sync_copy(zobuf, acc_k.at[pl.ds(off, stripe)])
    plsc.subcore_barrier()

    row0 = wid * rows_per_worker

    def _chunk(ci, carry):
        base = pl.multiple_of(row0 + ci * _K, 8)
        pltpu.sync_copy(src_hbm.at[pl.ds(base, _K)], src2d)
        pltpu.sync_copy(dst_hbm.at[pl.ds(base, _K)], dst2d)
        for j in range(_K):
            pltpu.sync_copy(ones_v, acc_do.at[src2d.at[j]], add=True)
            pltpu.sync_copy(ones_v, acc_di.at[dst2d.at[j]], add=True)
            for t in range(_ROW // 16):
                v = dst2d[j, pl.ds(t * 16, 16)]
                mrow[pl.ds(t * 16, 16)] = jnp.where(v == _TARGET, 1.0, 0.0)
            pltpu.sync_copy(mrow, acc_k.at[src2d.at[j]], add=True)
        return carry
    lax.fori_loop(0, chunks, _chunk, 0)
    plsc.subcore_barrier()

    obase = pl.multiple_of(cid * n_acc + off, 128)
    pltpu.sync_copy(acc_do.at[pl.ds(off, stripe)], zobuf)
    pltpu.sync_copy(zobuf, out_do.at[pl.ds(obase, stripe)])
    pltpu.sync_copy(acc_di.at[pl.ds(off, stripe)], zobuf)
    pltpu.sync_copy(zobuf, out_di.at[pl.ds(obase, stripe)])
    pltpu.sync_copy(acc_k.at[pl.ds(off, stripe)], zobuf)
    pltpu.sync_copy(zobuf, out_k.at[pl.ds(obase, stripe)])


def _segsum_body(n_acc, stripe, chunks, rows_per_worker,
                 src_hbm, dst_hbm, c_hbm, out_s1,
                 src2d, dst2d, cvals, zobuf, acc, sem):
    cid = lax.axis_index("c")
    sid = lax.axis_index("s")
    wid = sid * _NC + cid

    def _zero(i, carry):
        zobuf[pl.ds(i * 16, 16)] = jnp.zeros((16,), jnp.float32)
        return carry
    lax.fori_loop(0, stripe // 16, _zero, 0)

    off = pl.multiple_of(sid * stripe, 128)
    pltpu.sync_copy(zobuf, acc.at[pl.ds(off, stripe)])
    plsc.subcore_barrier()

    row0 = wid * rows_per_worker

    def _chunk(ci, carry):
        base = pl.multiple_of(row0 + ci * _K, 8)
        pltpu.sync_copy(src_hbm.at[pl.ds(base, _K)], src2d)
        pltpu.sync_copy(dst_hbm.at[pl.ds(base, _K)], dst2d)
        copies = []
        for j in range(_K):
            copies.append(
                pltpu.async_copy(c_hbm.at[src2d.at[j]], cvals.at[j], sem))
        for h in copies:
            h.wait()
        for j in range(_K):
            pltpu.sync_copy(cvals.at[j], acc.at[dst2d.at[j]], add=True)
        return carry
    lax.fori_loop(0, chunks, _chunk, 0)
    plsc.subcore_barrier()

    obase = pl.multiple_of(cid * n_acc + off, 128)
    pltpu.sync_copy(acc.at[pl.ds(off, stripe)], zobuf)
    pltpu.sync_copy(zobuf, out_s1.at[pl.ds(obase, stripe)])


def kernel(in_feat, edge_index, W0, b0, W1, b1):
    n = in_feat.shape[0]
    e = edge_index.shape[1]

    grain = _NW * _K * _ROW
    e_pad = -(-e // grain) * grain
    nrows = e_pad // _ROW
    rows_per_worker = nrows // _NW
    chunks = rows_per_worker // _K

    stripe = -(-(-(-n // _NS)) // 128) * 128
    n_acc = stripe * _NS
    if n_acc < n + 1:
        stripe += 128
        n_acc = stripe * _NS

    pad = jnp.full((e_pad - e,), n, dtype=jnp.int32)
    src = jnp.concatenate([edge_index[0], pad]).reshape(nrows, _ROW)
    dst = jnp.concatenate([edge_index[1], pad]).reshape(nrows, _ROW)

    stats = functools.partial(
        pl.kernel,
        out_type=[jax.ShapeDtypeStruct((_NC * n_acc,), jnp.float32)] * 3,
        mesh=_mesh(),
        scratch_types=[
            pltpu.VMEM((_K, _ROW), jnp.int32),
            pltpu.VMEM((_K, _ROW), jnp.int32),
            pltpu.VMEM((_ROW,), jnp.float32),
            pltpu.VMEM((_ROW,), jnp.float32),
            pltpu.VMEM((stripe,), jnp.float32),
            pltpu.VMEM_SHARED((n_acc,), jnp.float32),
            pltpu.VMEM_SHARED((n_acc,), jnp.float32),
            pltpu.VMEM_SHARED((n_acc,), jnp.float32),
        ],
    )(functools.partial(_edge_stats_body, n_acc, stripe, chunks,
                        rows_per_worker))
    do2, di2, k2 = stats(src, dst)

    do2 = do2.reshape(_NC, n_acc)
    di2 = di2.reshape(_NC, n_acc)
    k2 = k2.reshape(_NC, n_acc)
    deg_out = jnp.maximum(do2[0, :n] + do2[1, :n], 1.0)
    deg_in = jnp.maximum(di2[0, :n] + di2[1, :n], 1.0)
    k_cnt = k2[0, :n] + k2[1, :n]
    norm_src = lax.rsqrt(deg_out)
    norm_dst = lax.rsqrt(deg_in)

    c = in_feat[:, 0] * norm_src
    c_pad = jnp.concatenate([c, jnp.zeros((n_acc - n,), jnp.float32)])

    segsum = functools.partial(
        pl.kernel,
        out_type=jax.ShapeDtypeStruct((_NC * n_acc,), jnp.float32),
        mesh=_mesh(),
        scratch_types=[
            pltpu.VMEM((_K, _ROW), jnp.int32),
            pltpu.VMEM((_K, _ROW), jnp.int32),
            pltpu.VMEM((_K, _ROW), jnp.float32),
            pltpu.VMEM((stripe,), jnp.float32),
            pltpu.VMEM_SHARED((n_acc,), jnp.float32),
            pltpu.SemaphoreType.DMA,
        ],
    )(functools.partial(_segsum_body, n_acc, stripe, chunks,
                        rows_per_worker))
    s1_2 = segsum(src, dst, c_pad).reshape(_NC, n_acc)
    s1 = s1_2[0, :n] + s1_2[1, :n]

    a = s1 * norm_dst
    z = jax.nn.leaky_relu(a[:, None] * W0[0][None, :] + b0[None, :], 0.01)
    d = (z @ W1[:, 0]) * norm_src
    r = jnp.sum(k_cnt * d)
    out = jax.nn.leaky_relu(norm_dst[_TARGET] * r + b1, 0.01)
    return out
